# baseline (speedup 1.0000x reference)
import functools

import jax
import jax.numpy as jnp
from jax import lax
from jax.experimental import pallas as pl
from jax.experimental.pallas import tpu as pltpu
from jax.experimental.pallas import tpu_sc as plsc

B = 4096
V = 1000
D = 10000
NC, NS = 2, 16
NW = NC * NS
BPW = B // NW
C = 4
NCH = BPW // C


def _gather_grid(table, idx3):
    mesh = plsc.VectorSubcoreMesh(core_axis_name="c", subcore_axis_name="s")

    @functools.partial(
        pl.kernel,
        out_type=jax.ShapeDtypeStruct((B, D), jnp.float32),
        mesh=mesh,
        compiler_params=pltpu.CompilerParams(use_tc_tiling_on_sc=False),
        scratch_types=[
            pltpu.VMEM((NCH, C), jnp.int32),
            pltpu.VMEM((C, D), jnp.float32),
            pltpu.SemaphoreType.DMA,
        ],
    )
    def k(table_hbm, idx_hbm, out_hbm, idx_v, rows_v, sem):
        wid = lax.axis_index("s") * NC + lax.axis_index("c")
        pltpu.sync_copy(idx_hbm.at[wid], idx_v)

        def chunk(g, carry):
            pltpu.async_copy(table_hbm.at[idx_v.at[g]], rows_v, sem).wait()
            pltpu.sync_copy(rows_v, out_hbm.at[pl.ds(wid * BPW + g * C, C)])
            return carry

        lax.fori_loop(0, NCH, chunk, 0)

    return k(table, idx3)


def kernel(tokens, hdc_vocab):
    idx3 = tokens.astype(jnp.int32).reshape(NW, NCH, C)
    return _gather_grid(hdc_vocab, idx3)


# speedup vs baseline: 1.0440x; 1.0440x over previous
import functools

import jax
import jax.numpy as jnp
from jax import lax
from jax.experimental import pallas as pl
from jax.experimental.pallas import tpu as pltpu
from jax.experimental.pallas import tpu_sc as plsc

B = 4096
V = 1000
D = 10000
NC, NS = 2, 16
NW = NC * NS
BPW = B // NW
C = 4
NCH = BPW // C


def _gather_grid(table, idx3):
    mesh = plsc.VectorSubcoreMesh(core_axis_name="c", subcore_axis_name="s")

    @functools.partial(
        pl.kernel,
        out_type=jax.ShapeDtypeStruct((B, D), jnp.float32),
        mesh=mesh,
        compiler_params=pltpu.CompilerParams(use_tc_tiling_on_sc=False),
        scratch_types=[
            pltpu.VMEM((NCH, C), jnp.int32),
            pltpu.VMEM((C, D), jnp.float32),
            pltpu.VMEM((C, D), jnp.float32),
            pltpu.SemaphoreType.DMA,
            pltpu.SemaphoreType.DMA,
            pltpu.SemaphoreType.DMA,
            pltpu.SemaphoreType.DMA,
        ],
    )
    def k(table_hbm, idx_hbm, out_hbm, idx_v, buf0, buf1, g0, g1, o0, o1):
        wid = lax.axis_index("s") * NC + lax.axis_index("c")
        base = wid * BPW
        pltpu.sync_copy(idx_hbm.at[wid], idx_v)

        bufs = (buf0, buf1)
        gsems = (g0, g1)
        osems = (o0, o1)

        def gather(c, b):
            return pltpu.make_async_copy(table_hbm.at[idx_v.at[c]], bufs[b], gsems[b])

        def writeback(c, b):
            return pltpu.make_async_copy(
                bufs[b], out_hbm.at[pl.ds(base + c * C, C)], osems[b]
            )

        for b in range(2):
            d = gather(b, b)
            d.start()
            d.wait()
            writeback(b, b).start()

        @pl.loop(2, NCH, step=2)
        def _(g):
            for b in range(2):
                c = g + b
                writeback(c - 2, b).wait()
                d = gather(c, b)
                d.start()
                d.wait()
                writeback(c, b).start()

        for b in range(2):
            writeback(NCH - 2 + b, b).wait()

    return k(table, idx3)


def kernel(tokens, hdc_vocab):
    idx3 = tokens.astype(jnp.int32).reshape(NW, NCH, C)
    return _gather_grid(hdc_vocab, idx3)


# speedup vs baseline: 1.0719x; 1.0267x over previous
import functools

import jax
import jax.numpy as jnp
from jax import lax
from jax.experimental import pallas as pl
from jax.experimental.pallas import tpu as pltpu
from jax.experimental.pallas import tpu_sc as plsc

B = 4096
V = 1000
D = 10000
DP = 10240
NC, NS = 2, 16
NW = NC * NS
BPW = B // NW
C = 4
NCH = BPW // C


def _gather_grid(table3, idx3):
    mesh = plsc.VectorSubcoreMesh(core_axis_name="c", subcore_axis_name="s")

    @functools.partial(
        pl.kernel,
        out_type=jax.ShapeDtypeStruct((B, DP // 128, 128), jnp.float32),
        mesh=mesh,
        compiler_params=pltpu.CompilerParams(use_tc_tiling_on_sc=False),
        scratch_types=[
            pltpu.VMEM((NCH, C), jnp.int32),
            pltpu.VMEM((C, DP // 128, 128), jnp.float32),
            pltpu.VMEM((C, DP // 128, 128), jnp.float32),
            pltpu.SemaphoreType.DMA,
            pltpu.SemaphoreType.DMA,
            pltpu.SemaphoreType.DMA,
            pltpu.SemaphoreType.DMA,
        ],
    )
    def k(table_hbm, idx_hbm, out_hbm, idx_v, buf0, buf1, g0, g1, o0, o1):
        wid = lax.axis_index("s") * NC + lax.axis_index("c")
        base = wid * BPW
        pltpu.sync_copy(idx_hbm.at[wid], idx_v)

        bufs = (buf0, buf1)
        gsems = (g0, g1)
        osems = (o0, o1)

        def gather(c, b):
            return pltpu.make_async_copy(
                table_hbm.at[idx_v.at[c]], bufs[b], gsems[b]
            )

        def writeback_rows(c, b):
            return pltpu.make_async_copy(
                bufs[b],
                out_hbm.at[pl.ds(base + c * C, C)],
                osems[b],
            )

        for b in range(2):
            d = gather(b, b)
            d.start()
            d.wait()
            writeback_rows(b, b).start()

        @pl.loop(2, NCH, step=2)
        def _(g):
            for b in range(2):
                c = g + b
                writeback_rows(c - 2, b).wait()
                d = gather(c, b)
                d.start()
                d.wait()
                writeback_rows(c, b).start()

        for b in range(2):
            writeback_rows(NCH - 2 + b, b).wait()

    return k(table3, idx3)


def kernel(tokens, hdc_vocab):
    table3 = jnp.pad(hdc_vocab, ((0, 0), (0, DP - D))).reshape(V, DP // 128, 128)
    idx3 = tokens.astype(jnp.int32).reshape(NW, NCH, C)
    out3 = _gather_grid(table3, idx3)
    return out3.reshape(B, DP)[:, :D]


# speedup vs baseline: 1.4151x; 1.3201x over previous
import functools

import jax
import jax.numpy as jnp
from jax import lax
from jax.experimental import pallas as pl
from jax.experimental.pallas import tpu as pltpu
from jax.experimental.pallas import tpu_sc as plsc

B = 4096
V = 1000
D = 10000
DP = 10112
NC, NS = 2, 16
NW = NC * NS
BPW = B // NW
C = 8
NCH = BPW // C


def _gather_grid(table_p, idx3):
    mesh = plsc.VectorSubcoreMesh(core_axis_name="c", subcore_axis_name="s")

    @functools.partial(
        pl.kernel,
        out_type=jax.ShapeDtypeStruct((B, D), jnp.float32),
        mesh=mesh,
        compiler_params=pltpu.CompilerParams(use_tc_tiling_on_sc=True),
        scratch_types=[
            pltpu.VMEM((NCH, C), jnp.int32),
            pltpu.VMEM((C, DP), jnp.float32),
            pltpu.VMEM((C, 16), jnp.float32),
            pltpu.SemaphoreType.DMA,
            pltpu.SemaphoreType.DMA,
        ],
    )
    def k(table_hbm, idx_hbm, out_hbm, idx_v, buf, tailbuf, gsem, osem):
        wid = lax.axis_index("s") * NC + lax.axis_index("c")
        base = wid * BPW
        pltpu.sync_copy(idx_hbm.at[wid], idx_v)

        @pl.loop(0, NCH)
        def _(c):
            pltpu.async_copy(table_hbm.at[idx_v.at[c]], buf, gsem).wait()
            rows = pl.ds(base + c * C, C)
            pltpu.async_copy(
                buf.at[:, pl.ds(0, D - 16)],
                out_hbm.at[rows, pl.ds(0, D - 16)],
                osem,
            )
            for r in range(C):
                tailbuf[r, :] = buf[r, pl.ds(D - 16, 16)]
            pltpu.async_copy(
                tailbuf,
                out_hbm.at[rows, pl.ds(D - 16, 16)],
                osem,
            )
            pltpu.make_async_copy(
                buf.at[:, pl.ds(0, D - 16)], out_hbm.at[rows, pl.ds(0, D - 16)], osem
            ).wait()
            pltpu.make_async_copy(
                tailbuf, out_hbm.at[rows, pl.ds(D - 16, 16)], osem
            ).wait()

    return k(table_p, idx3)


def kernel(tokens, hdc_vocab):
    table_p = jnp.pad(hdc_vocab, ((0, 0), (0, DP - D)))
    idx3 = tokens.astype(jnp.int32).reshape(NW, NCH, C)
    return _gather_grid(table_p, idx3)


# speedup vs baseline: 1.4462x; 1.0220x over previous
import functools

import jax
import jax.numpy as jnp
from jax import lax
from jax.experimental import pallas as pl
from jax.experimental.pallas import tpu as pltpu
from jax.experimental.pallas import tpu_sc as plsc

B = 4096
V = 1000
D = 10000
DP = 10112
WL = 5120
WR = DP - WL
WRM = 4864
NC, NS = 2, 16
NW = NC * NS
BPW = B // NW
C = 8
NCH = BPW // C


def _gather_grid(table_p, idx3):
    mesh = plsc.VectorSubcoreMesh(core_axis_name="c", subcore_axis_name="s")

    @functools.partial(
        pl.kernel,
        out_type=jax.ShapeDtypeStruct((B, D), jnp.float32),
        mesh=mesh,
        compiler_params=pltpu.CompilerParams(use_tc_tiling_on_sc=True),
        scratch_types=[
            pltpu.VMEM((NCH, C), jnp.int32),
            pltpu.VMEM((C, WL), jnp.float32),
            pltpu.VMEM((C, WR), jnp.float32),
            pltpu.VMEM((C, 16), jnp.float32),
            pltpu.SemaphoreType.DMA,
            pltpu.SemaphoreType.DMA,
            pltpu.SemaphoreType.DMA,
            pltpu.SemaphoreType.DMA,
        ],
    )
    def k(table_hbm, idx_hbm, out_hbm, idx_v, bufL, bufR, tailbuf,
          gsemL, gsemR, osemL, osemR):
        wid = lax.axis_index("s") * NC + lax.axis_index("c")
        base = wid * BPW
        pltpu.sync_copy(idx_hbm.at[wid], idx_v)

        def gatherL(c):
            return pltpu.make_async_copy(
                table_hbm.at[idx_v.at[c], pl.ds(0, WL)], bufL, gsemL
            )

        def gatherR(c):
            return pltpu.make_async_copy(
                table_hbm.at[idx_v.at[c], pl.ds(WL, WR)], bufR, gsemR
            )

        def wbL(c):
            return pltpu.make_async_copy(
                bufL, out_hbm.at[pl.ds(base + c * C, C), pl.ds(0, WL)], osemL
            )

        def wbRmain(c):
            return pltpu.make_async_copy(
                bufR.at[:, pl.ds(0, WRM)],
                out_hbm.at[pl.ds(base + c * C, C), pl.ds(WL, WRM)],
                osemR,
            )

        def wbTail(c):
            return pltpu.make_async_copy(
                tailbuf,
                out_hbm.at[pl.ds(base + c * C, C), pl.ds(D - 16, 16)],
                osemR,
            )

        def body(c, last):
            gatherL(c).wait()
            wbL(c).start()
            d = gatherR(c)
            d.start()
            d.wait()
            wbRmain(c).start()
            for r in range(C):
                tailbuf[r, :] = bufR[r, pl.ds(WRM, 16)]
            wbTail(c).start()
            if not last:
                wbL(c).wait()
                gatherL(c + 1).start()
            wbRmain(c).wait()
            wbTail(c).wait()

        gatherL(0).start()

        @pl.loop(0, NCH - 1)
        def _(c):
            body(c, last=False)

        body(NCH - 1, last=True)
        wbL(NCH - 1).wait()

    return k(table_p, idx3)


def kernel(tokens, hdc_vocab):
    table_p = jnp.pad(hdc_vocab, ((0, 0), (0, DP - D)))
    idx3 = tokens.astype(jnp.int32).reshape(NW, NCH, C)
    return _gather_grid(table_p, idx3)


# speedup vs baseline: 1.5771x; 1.0905x over previous
import functools

import jax
import jax.numpy as jnp
from jax import lax
from jax.experimental import pallas as pl
from jax.experimental.pallas import tpu as pltpu
from jax.experimental.pallas import tpu_sc as plsc

B = 4096
V = 1000
D = 10000
DA = 9984
WL = 5120
WR = DA - WL
NC, NS = 2, 16
NW = NC * NS
BPW = B // NW
C = 8
NCH = BPW // C


def _gather_grid(table, tail_t, idx3):
    mesh = plsc.VectorSubcoreMesh(core_axis_name="c", subcore_axis_name="s")

    @functools.partial(
        pl.kernel,
        out_type=jax.ShapeDtypeStruct((B, D), jnp.float32),
        mesh=mesh,
        compiler_params=pltpu.CompilerParams(use_tc_tiling_on_sc=True),
        scratch_types=[
            pltpu.VMEM((NCH, C), jnp.int32),
            pltpu.VMEM((C, WL), jnp.float32),
            pltpu.VMEM((C, WR), jnp.float32),
            pltpu.VMEM((C, 128), jnp.float32),
            pltpu.VMEM((C, 16), jnp.float32),
            pltpu.SemaphoreType.DMA,
            pltpu.SemaphoreType.DMA,
            pltpu.SemaphoreType.DMA,
            pltpu.SemaphoreType.DMA,
        ],
    )
    def k(table_hbm, tail_hbm, idx_hbm, out_hbm, idx_v, bufL, bufR, tailg,
          tailbuf, gsemL, gsemR, osemL, osemR):
        wid = lax.axis_index("s") * NC + lax.axis_index("c")
        base = wid * BPW
        pltpu.sync_copy(idx_hbm.at[wid], idx_v)

        def gatherL(c):
            return pltpu.make_async_copy(
                table_hbm.at[idx_v.at[c], pl.ds(0, WL)], bufL, gsemL
            )

        def gatherR(c):
            return pltpu.make_async_copy(
                table_hbm.at[idx_v.at[c], pl.ds(WL, WR)], bufR, gsemR
            )

        def gatherT(c):
            return pltpu.make_async_copy(
                tail_hbm.at[idx_v.at[c]], tailg, gsemR
            )

        def wbL(c):
            return pltpu.make_async_copy(
                bufL, out_hbm.at[pl.ds(base + c * C, C), pl.ds(0, WL)], osemL
            )

        def wbR(c):
            return pltpu.make_async_copy(
                bufR, out_hbm.at[pl.ds(base + c * C, C), pl.ds(WL, WR)], osemR
            )

        def wbTail(c):
            return pltpu.make_async_copy(
                tailbuf,
                out_hbm.at[pl.ds(base + c * C, C), pl.ds(DA, 16)],
                osemR,
            )

        def body(c, last):
            gatherL(c).wait()
            wbL(c).start()
            gatherR(c).start()
            gatherT(c).start()
            gatherR(c).wait()
            gatherT(c).wait()
            wbR(c).start()
            for r in range(C):
                tailbuf[r, :] = tailg[r, pl.ds(0, 16)]
            wbTail(c).start()
            if not last:
                wbL(c).wait()
                gatherL(c + 1).start()
            wbR(c).wait()
            wbTail(c).wait()

        gatherL(0).start()

        @pl.loop(0, NCH - 1)
        def _(c):
            body(c, last=False)

        body(NCH - 1, last=True)
        wbL(NCH - 1).wait()

    return k(table, tail_t, idx3)


def kernel(tokens, hdc_vocab):
    tail_t = jnp.pad(hdc_vocab[:, DA:], ((0, 0), (0, 128 - (D - DA))))
    idx3 = tokens.astype(jnp.int32).reshape(NW, NCH, C)
    return _gather_grid(hdc_vocab, tail_t, idx3)
